# baseline (speedup 1.0000x reference)
import functools

import numpy as np

import jax
import jax.numpy as jnp
from jax import lax
from jax.experimental import pallas as pl
from jax.experimental.pallas import tpu as pltpu
from jax.experimental.pallas import tpu_sc as plsc

_NUM_RESIDUES = 100000
_NUM_MODS = 1000
_D_RES = 64
_D_MOD = 32
_D_MODEL = 128
_BATCH = 4096
_SEQ = 200
_N_TOKENS = _BATCH * _SEQ

_NC = 2
_NS = 16
_NW = _NC * _NS
_PER_W = _N_TOKENS // _NW
_CHUNK = 64
_NCHUNK = _PER_W // _CHUNK
_NBUF = 4


def _interleave_perm():
    perm = np.empty(_D_MODEL, np.int32)
    for j in range(_D_MODEL // 32):
        for i in range(16):
            perm[32 * j + 2 * i] = 32 * j + i
            perm[32 * j + 2 * i + 1] = 32 * j + 16 + i
    return perm


def _pack_rows_i32(y):
    return jax.lax.bitcast_convert_type(
        y.reshape(y.shape[0], _D_MODEL // 2, 2), jnp.int32)


def _proj_res_body(r_ref, w_ref, o_ref):
    o_ref[...] = jnp.dot(r_ref[...], w_ref[...],
                         preferred_element_type=jnp.float32).astype(jnp.bfloat16)


def _project_residue(table, w_res):
    blk = 2000
    grid = _NUM_RESIDUES // blk
    return pl.pallas_call(
        _proj_res_body,
        grid=(grid,),
        in_specs=[
            pl.BlockSpec((blk, _D_RES), lambda i: (i, 0)),
            pl.BlockSpec((_D_RES, _D_MODEL), lambda i: (0, 0)),
        ],
        out_specs=pl.BlockSpec((blk, _D_MODEL), lambda i: (i, 0)),
        out_shape=jax.ShapeDtypeStruct((_NUM_RESIDUES, _D_MODEL), jnp.bfloat16),
    )(table, w_res)


def _proj_mod_body(m_ref, w_ref, b_ref, o_ref):
    o_ref[...] = (jnp.dot(m_ref[...], w_ref[...],
                          preferred_element_type=jnp.float32)
                  + b_ref[...]).astype(jnp.bfloat16)


def _project_mod(table, w_mod, bias):
    return pl.pallas_call(
        _proj_mod_body,
        out_shape=jax.ShapeDtypeStruct((_NUM_MODS, _D_MODEL), jnp.bfloat16),
    )(table, w_mod, bias.reshape(1, _D_MODEL))


def _sc_body(ri_hbm, mi_hbm, rp_hbm, mp_hbm, out_hbm,
             ri_v, mi_v, rows_r, rows_m, outs, sems_g, sems_w):
    wid = lax.axis_index("s") * _NC + lax.axis_index("c")
    base0 = wid * _PER_W

    pltpu.sync_copy(ri_hbm.at[pl.ds(base0, _PER_W)], ri_v)
    pltpu.sync_copy(mi_hbm.at[pl.ds(base0, _PER_W)], mi_v)

    def issue_gather(cur, b):
        off = cur * _CHUNK
        pltpu.async_copy(rp_hbm.at[ri_v.at[pl.ds(off, _CHUNK)]], rows_r[b], sems_g[b])
        pltpu.async_copy(mp_hbm.at[mi_v.at[pl.ds(off, _CHUNK)]], rows_m[b], sems_g[b])

    def wait_gather(cur, b):
        off = cur * _CHUNK
        pltpu.make_async_copy(rp_hbm.at[ri_v.at[pl.ds(off, _CHUNK)]], rows_r[b], sems_g[b]).wait()
        pltpu.make_async_copy(mp_hbm.at[mi_v.at[pl.ds(off, _CHUNK)]], rows_m[b], sems_g[b]).wait()

    def wait_writeback(b):
        pltpu.make_async_copy(outs[b], out_hbm.at[pl.ds(base0, _CHUNK)], sems_w[b]).wait()

    issue_gather(0, 0)
    issue_gather(1, 1)

    def outer(i0, carry):
        for b in range(_NBUF):
            cur = _NBUF * i0 + b
            pf = cur + 2
            pb = (b + 2) % _NBUF

            @pl.when(pf < _NCHUNK)
            def _():
                @pl.when(cur >= 2)
                def _():
                    wait_writeback(pb)
                issue_gather(pf, pb)

            wait_gather(cur, b)

            rr = rows_r[b]
            rm = rows_m[b]
            ov = outs[b]

            def add_row(r, c):
                for k in range(_D_MODEL // 32):
                    sl = pl.ds(16 * k, 16)
                    pr = plsc.bitcast(rr[r, sl], jnp.bfloat16)
                    pm = plsc.bitcast(rm[r, sl], jnp.bfloat16)
                    ar, br = plsc.unpack(pr, format=plsc.PackFormat.INTERLEAVED)
                    am, bm = plsc.unpack(pm, format=plsc.PackFormat.INTERLEAVED)
                    ov[r, pl.ds(32 * k, 16)] = ar + am
                    ov[r, pl.ds(32 * k + 16, 16)] = br + bm
                return c

            lax.fori_loop(0, _CHUNK, add_row, 0, unroll=2)
            pltpu.async_copy(ov, out_hbm.at[pl.ds(base0 + cur * _CHUNK, _CHUNK)],
                             sems_w[b])
        return carry

    lax.fori_loop(0, _NCHUNK // _NBUF, outer, 0)
    for b in range(_NBUF):
        wait_writeback(b)


def _sc_gather_add(ri_flat, mi_flat, rp, mp):
    mesh = plsc.VectorSubcoreMesh(core_axis_name="c", subcore_axis_name="s")
    f = pl.kernel(
        _sc_body,
        out_type=jax.ShapeDtypeStruct((_N_TOKENS, _D_MODEL), jnp.float32),
        mesh=mesh,
        compiler_params=pltpu.CompilerParams(needs_layout_passes=False,
                                             use_tc_tiling_on_sc=False),
        scratch_types=[
            pltpu.VMEM((_PER_W,), jnp.int32),
            pltpu.VMEM((_PER_W,), jnp.int32),
            [pltpu.VMEM((_CHUNK, _D_MODEL // 2), jnp.int32) for _ in range(_NBUF)],
            [pltpu.VMEM((_CHUNK, _D_MODEL // 2), jnp.int32) for _ in range(_NBUF)],
            [pltpu.VMEM((_CHUNK, _D_MODEL), jnp.float32) for _ in range(_NBUF)],
            [pltpu.SemaphoreType.DMA for _ in range(_NBUF)],
            [pltpu.SemaphoreType.DMA for _ in range(_NBUF)],
        ],
    )
    return f(ri_flat, mi_flat, rp, mp)


def kernel(residue_indices, modification_indices, residue_table,
           modification_table, proj_kernel, proj_bias):
    perm = _interleave_perm()
    w_perm = proj_kernel[:, perm]
    b_perm = proj_bias[perm]
    w_res = w_perm[:_D_RES]
    w_mod = w_perm[_D_RES:]
    rp = _pack_rows_i32(_project_residue(residue_table, w_res))
    mp = _pack_rows_i32(_project_mod(modification_table, w_mod, b_perm))
    ri_flat = residue_indices.reshape(_N_TOKENS)
    mi_flat = modification_indices.reshape(_N_TOKENS)
    out_flat = _sc_gather_add(ri_flat, mi_flat, rp, mp)
    return out_flat.reshape(_BATCH, _SEQ, _D_MODEL)


# speedup vs baseline: 1.8147x; 1.8147x over previous
import functools

import numpy as np

import jax
import jax.numpy as jnp
from jax import lax
from jax.experimental import pallas as pl
from jax.experimental.pallas import tpu as pltpu
from jax.experimental.pallas import tpu_sc as plsc

_NUM_RESIDUES = 100000
_NUM_MODS = 1000
_D_RES = 64
_D_MOD = 32
_D_MODEL = 128
_BATCH = 4096
_SEQ = 200
_N_TOKENS = _BATCH * _SEQ

_NC = 2
_NS = 16
_NW = _NC * _NS
_PER_W = _N_TOKENS // _NW
_CHUNK = 64
_NCHUNK = _PER_W // _CHUNK
_NBUF = 4


def _proj_res_body(r_ref, w_ref, o_ref):
    o_ref[...] = jnp.dot(r_ref[...], w_ref[...],
                         preferred_element_type=jnp.float32)


def _project_residue(table, w_res):
    blk = 2000
    grid = _NUM_RESIDUES // blk
    return pl.pallas_call(
        _proj_res_body,
        grid=(grid,),
        in_specs=[
            pl.BlockSpec((blk, _D_RES), lambda i: (i, 0)),
            pl.BlockSpec((_D_RES, _D_MODEL), lambda i: (0, 0)),
        ],
        out_specs=pl.BlockSpec((blk, _D_MODEL), lambda i: (i, 0)),
        out_shape=jax.ShapeDtypeStruct((_NUM_RESIDUES, _D_MODEL), jnp.float32),
    )(table, w_res)


def _proj_mod_body(m_ref, w_ref, b_ref, o_ref):
    o_ref[...] = (jnp.dot(m_ref[...], w_ref[...],
                          preferred_element_type=jnp.float32)
                  + b_ref[...])


def _project_mod(table, w_mod, bias):
    return pl.pallas_call(
        _proj_mod_body,
        out_shape=jax.ShapeDtypeStruct((_NUM_MODS, _D_MODEL), jnp.float32),
    )(table, w_mod, bias.reshape(1, _D_MODEL))


def _sc_body(ri_hbm, mi_hbm, rp_hbm, mp_hbm, out_hbm,
             ri_v, mi_v, rows_r, rows_m, sems_g, sems_w):
    wid = lax.axis_index("s") * _NC + lax.axis_index("c")
    base0 = wid * _PER_W

    pltpu.sync_copy(ri_hbm.at[pl.ds(base0, _PER_W)], ri_v)
    pltpu.sync_copy(mi_hbm.at[pl.ds(base0, _PER_W)], mi_v)

    def issue_gather(cur, b):
        off = cur * _CHUNK
        pltpu.async_copy(rp_hbm.at[ri_v.at[pl.ds(off, _CHUNK)]], rows_r[b], sems_g[b])
        pltpu.async_copy(mp_hbm.at[mi_v.at[pl.ds(off, _CHUNK)]], rows_m[b], sems_g[b])

    def wait_gather(cur, b):
        off = cur * _CHUNK
        pltpu.make_async_copy(rp_hbm.at[ri_v.at[pl.ds(off, _CHUNK)]], rows_r[b], sems_g[b]).wait()
        pltpu.make_async_copy(mp_hbm.at[mi_v.at[pl.ds(off, _CHUNK)]], rows_m[b], sems_g[b]).wait()

    def wait_writeback(b):
        pltpu.make_async_copy(rows_r[b], out_hbm.at[pl.ds(base0, _CHUNK)], sems_w[b]).wait()

    issue_gather(0, 0)
    issue_gather(1, 1)

    def outer(i0, carry):
        for b in range(_NBUF):
            cur = _NBUF * i0 + b
            pf = cur + 2
            pb = (b + 2) % _NBUF

            @pl.when(pf < _NCHUNK)
            def _():
                @pl.when(cur >= 2)
                def _():
                    wait_writeback(pb)
                issue_gather(pf, pb)

            wait_gather(cur, b)

            rr = rows_r[b]
            rm = rows_m[b]

            def add_row(r, c):
                for k in range(_D_MODEL // 16):
                    sl = pl.ds(16 * k, 16)
                    plsc.addupdate(rr.at[r, sl], rm[r, sl])
                return c

            lax.fori_loop(0, _CHUNK, add_row, 0, unroll=2)
            pltpu.async_copy(rr, out_hbm.at[pl.ds(base0 + cur * _CHUNK, _CHUNK)],
                             sems_w[b])
        return carry

    lax.fori_loop(0, _NCHUNK // _NBUF, outer, 0)
    for b in range(_NBUF):
        wait_writeback(b)


def _sc_gather_add(ri_flat, mi_flat, rp, mp):
    mesh = plsc.VectorSubcoreMesh(core_axis_name="c", subcore_axis_name="s")
    f = pl.kernel(
        _sc_body,
        out_type=jax.ShapeDtypeStruct((_N_TOKENS, _D_MODEL), jnp.float32),
        mesh=mesh,
        scratch_types=[
            pltpu.VMEM((_PER_W,), jnp.int32),
            pltpu.VMEM((_PER_W,), jnp.int32),
            [pltpu.VMEM((_CHUNK, _D_MODEL), jnp.float32) for _ in range(_NBUF)],
            [pltpu.VMEM((_CHUNK, _D_MODEL), jnp.float32) for _ in range(_NBUF)],
            [pltpu.SemaphoreType.DMA for _ in range(_NBUF)],
            [pltpu.SemaphoreType.DMA for _ in range(_NBUF)],
        ],
    )
    return f(ri_flat, mi_flat, rp, mp)


def kernel(residue_indices, modification_indices, residue_table,
           modification_table, proj_kernel, proj_bias):
    w_res = proj_kernel[:_D_RES]
    w_mod = proj_kernel[_D_RES:]
    rp = _project_residue(residue_table, w_res)
    mp = _project_mod(modification_table, w_mod, proj_bias)
    ri_flat = residue_indices.reshape(_N_TOKENS)
    mi_flat = modification_indices.reshape(_N_TOKENS)
    out_flat = _sc_gather_add(ri_flat, mi_flat, rp, mp)
    return out_flat.reshape(_BATCH, _SEQ, _D_MODEL)
